# baseline (speedup 1.0000x reference)
import functools

import jax
import jax.numpy as jnp
from jax import lax
from jax.experimental import pallas as pl
from jax.experimental.pallas import tpu as pltpu
from jax.experimental.pallas import tpu_sc as plsc

_BS = 64
_SEQ = 200
_VOCAB = 1000
_N_OOV = 50
_D = 1024
_PAD = 0

_N_TOKENS = _BS * _SEQ
_LANES = 16
_CHUNK = 40
_OCHUNK = 48


@functools.cache
def _build_gather():
    info = plsc.get_sparse_core_info()
    nc, ns = info.num_cores, info.num_subcores
    nw = nc * ns
    per_w = _N_TOKENS // nw
    assert per_w % _CHUNK == 0 and per_w % _LANES == 0
    n_chunks = per_w // _CHUNK
    n_vec = per_w // _LANES
    n_ochunks = -(-per_w // _OCHUNK)
    olist = n_ochunks * _OCHUNK
    mesh = plsc.VectorSubcoreMesh(
        core_axis_name="c", subcore_axis_name="s")

    @functools.partial(
        pl.kernel,
        out_type=jax.ShapeDtypeStruct((_N_TOKENS, _D), jnp.float32),
        mesh=mesh,
        scratch_types=[
            pltpu.VMEM((per_w,), jnp.int32),
            pltpu.SMEM((per_w,), jnp.int32),
            pltpu.SMEM((olist,), jnp.int32),
            pltpu.SMEM((olist,), jnp.int32),
            pltpu.VMEM((n_ochunks, _OCHUNK), jnp.int32),
        ] + [
            pltpu.VMEM((_OCHUNK,), jnp.int32) for _ in range(n_ochunks)
        ] + [
            pltpu.VMEM((_LANES,), jnp.int32),
            pltpu.VMEM((_OCHUNK, _D), jnp.float32),
            pltpu.SemaphoreType.DMA,
            pltpu.SemaphoreType.DMA,
        ],
    )
    def gather_kernel(fixed_hbm, oov_hbm, tokens_hbm, out_hbm,
                      tok_v, tok_s, oidx_s, opos_s, idx2_v, *rest):
        pos_refs = rest[:n_ochunks]
        fix_v, rows_v, g_sem, o_sem = rest[n_ochunks:]
        wid = lax.axis_index("s") * nc + lax.axis_index("c")
        base = wid * per_w
        lanes = lax.iota(jnp.int32, _LANES)

        pltpu.sync_copy(tokens_hbm.at[pl.ds(base, per_w)], tok_v)
        for g in range(n_vec):
            v = tok_v[pl.ds(g * _LANES, _LANES)]
            for k in range(_LANES):
                tok_s[g * _LANES + k] = v[k]

        t0 = tok_s[0]
        t16 = tok_v[pl.ds(0, _LANES)]
        fix_v[...] = (jnp.clip(t16 - _VOCAB, 0, _N_OOV - 1)
                      + wid * (2 * _N_OOV))
        woff = wid * (2 * _N_OOV)
        pad_oidx = jnp.clip(t0 - _VOCAB, 0, _N_OOV - 1) + woff

        def compact_body(i, n_o):
            t = tok_s[i]
            is_o = t >= _VOCAB
            orow = t - _VOCAB + woff + jnp.where(i >= _SEQ, _N_OOV, 0)
            oidx_s[n_o] = jnp.where(is_o, orow, pad_oidx)
            opos_s[n_o] = jnp.where(is_o, base + i, base)
            return n_o + jnp.where(is_o, 1, 0).astype(jnp.int32)

        n_o = lax.fori_loop(0, per_w, compact_body, jnp.int32(0))

        for i in range(n_vec):
            t = tok_v[pl.ds(i * _LANES, _LANES)]
            tok_v[pl.ds(i * _LANES, _LANES)] = jnp.minimum(t, _VOCAB - 1)

        for c in range(n_chunks):
            pltpu.async_copy(
                fixed_hbm.at[tok_v.at[pl.ds(c * _CHUNK, _CHUNK)]],
                rows_v.at[pl.ds(0, _CHUNK)], g_sem).wait()
            pltpu.async_copy(
                rows_v.at[pl.ds(0, _CHUNK)],
                out_hbm.at[pl.ds(base + c * _CHUNK, _CHUNK)], o_sem).wait()

        for c in range(n_ochunks):
            @pl.when(n_o > c * _OCHUNK)
            def _(c=c):
                for k in range(_OCHUNK // _LANES):
                    iacc = jnp.full((_LANES,), pad_oidx, dtype=jnp.int32)
                    pacc = jnp.full((_LANES,), base, dtype=jnp.int32)
                    for j in range(_LANES):
                        slot = c * _OCHUNK + k * _LANES + j
                        in_list = slot < n_o
                        ival = jnp.where(in_list, oidx_s[slot], pad_oidx)
                        pval = jnp.where(in_list, opos_s[slot], base)
                        iacc = jnp.where(
                            lanes == j, jnp.full((_LANES,), ival), iacc)
                        pacc = jnp.where(
                            lanes == j, jnp.full((_LANES,), pval), pacc)
                    idx2_v[c, pl.ds(k * _LANES, _LANES)] = iacc
                    pos_refs[c][pl.ds(k * _LANES, _LANES)] = pacc
                pltpu.async_copy(
                    oov_hbm.at[idx2_v.at[c]], rows_v, g_sem).wait()
                pltpu.async_copy(
                    rows_v, out_hbm.at[pos_refs[c]], o_sem).wait()

        pltpu.async_copy(
            fixed_hbm.at[tok_v.at[pl.ds(0, 8)]],
            rows_v.at[pl.ds(0, 8)], g_sem).wait()
        pltpu.async_copy(
            oov_hbm.at[fix_v.at[pl.ds(0, 8)]],
            rows_v.at[pl.ds(8, 8)], g_sem).wait()
        for r in range(8):
            @pl.when(tok_s[r] >= _VOCAB)
            def _(r=r):
                for m in range(_D // _LANES):
                    rows_v[r, pl.ds(m * _LANES, _LANES)] = rows_v[
                        8 + r, pl.ds(m * _LANES, _LANES)]
        pltpu.async_copy(
            rows_v.at[pl.ds(0, 8)],
            out_hbm.at[pl.ds(base, 8)], o_sem).wait()

    return gather_kernel


def kernel(tokens, oov_features, fixed_weights):
    tokens_i32 = tokens.astype(jnp.int32)
    flat_tokens = tokens_i32.reshape(_N_TOKENS)
    oov_flat = oov_features.reshape(_BS * _N_OOV, _D)
    features = _build_gather()(
        fixed_weights, oov_flat, flat_tokens).reshape(_BS, _SEQ, _D)
    padding_mask = (tokens == _PAD)[:, None, None, :]
    sequential_mask = jnp.triu(jnp.ones((_SEQ, _SEQ), dtype=bool), k=1)
    return (features, (padding_mask, sequential_mask))


# speedup vs baseline: 1.6241x; 1.6241x over previous
import functools

import jax
import jax.numpy as jnp
from jax import lax
from jax.experimental import pallas as pl
from jax.experimental.pallas import tpu as pltpu
from jax.experimental.pallas import tpu_sc as plsc

_BS = 64
_SEQ = 200
_VOCAB = 1000
_N_OOV = 50
_D = 1024
_PAD = 0

_N_TOKENS = _BS * _SEQ
_LANES = 16
_CHUNK = 40


@functools.cache
def _build_gather():
    info = plsc.get_sparse_core_info()
    nc, ns = info.num_cores, info.num_subcores
    nw = nc * ns
    per_w = _N_TOKENS // nw
    assert per_w % _CHUNK == 0 and per_w % _LANES == 0
    n_chunks = per_w // _CHUNK
    n_vec = per_w // _LANES
    mesh = plsc.VectorSubcoreMesh(
        core_axis_name="c", subcore_axis_name="s")

    @functools.partial(
        pl.kernel,
        out_type=jax.ShapeDtypeStruct((_N_TOKENS, _D), jnp.float32),
        mesh=mesh,
        scratch_types=[
            pltpu.VMEM((per_w,), jnp.int32),
            pltpu.VMEM((_CHUNK, _D), jnp.float32),
            pltpu.VMEM((_CHUNK, _D), jnp.float32),
            pltpu.SemaphoreType.DMA,
            pltpu.SemaphoreType.DMA,
            pltpu.SemaphoreType.DMA,
            pltpu.SemaphoreType.DMA,
        ],
    )
    def gather_kernel(table_hbm, rows_hbm, out_hbm, idx_v,
                      rows_a, rows_b, g_sem_a, g_sem_b, o_sem_a, o_sem_b):
        wid = lax.axis_index("s") * nc + lax.axis_index("c")
        base = wid * per_w
        pltpu.sync_copy(rows_hbm.at[pl.ds(base, per_w)], idx_v)
        bufs = (rows_a, rows_b)
        g_sems = (g_sem_a, g_sem_b)
        o_sems = (o_sem_a, o_sem_b)

        def start_gather(c):
            return pltpu.async_copy(
                table_hbm.at[idx_v.at[pl.ds(c * _CHUNK, _CHUNK)]],
                bufs[c % 2], g_sems[c % 2])

        def start_out(c):
            return pltpu.async_copy(
                bufs[c % 2], out_hbm.at[pl.ds(base + c * _CHUNK, _CHUNK)],
                o_sems[c % 2])

        gather_d = [None] * n_chunks
        out_d = [None] * n_chunks
        for c in range(n_chunks):
            if c >= 2:
                out_d[c - 2].wait()
            gather_d[c] = start_gather(c)
            if c >= 1:
                gather_d[c - 1].wait()
                out_d[c - 1] = start_out(c - 1)
        gather_d[n_chunks - 1].wait()
        out_d[n_chunks - 1] = start_out(n_chunks - 1)
        if n_chunks >= 2:
            out_d[n_chunks - 2].wait()
        out_d[n_chunks - 1].wait()

    return gather_kernel


def kernel(tokens, oov_features, fixed_weights):
    tokens_i32 = tokens.astype(jnp.int32)
    flat_tokens = tokens_i32.reshape(_N_TOKENS)
    table = jnp.concatenate(
        [fixed_weights, oov_features.reshape(_BS * _N_OOV, _D)], axis=0)
    boff = jnp.arange(_BS, dtype=jnp.int32)[:, None] * _N_OOV
    rows = jnp.where(tokens_i32 < _VOCAB, tokens_i32,
                     tokens_i32 + boff).reshape(_N_TOKENS)
    features = _build_gather()(table, rows).reshape(_BS, _SEQ, _D)
    padding_mask = (tokens == _PAD)[:, None, None, :]
    sequential_mask = jnp.triu(jnp.ones((_SEQ, _SEQ), dtype=bool), k=1)
    return (features, (padding_mask, sequential_mask))


# speedup vs baseline: 1.6252x; 1.0007x over previous
import functools

import jax
import jax.numpy as jnp
from jax import lax
from jax.experimental import pallas as pl
from jax.experimental.pallas import tpu as pltpu
from jax.experimental.pallas import tpu_sc as plsc

_BS = 64
_SEQ = 200
_VOCAB = 1000
_N_OOV = 50
_D = 1024
_PAD = 0

_N_TOKENS = _BS * _SEQ
_LANES = 16
_CHUNK = 40


@functools.cache
def _build_gather():
    info = plsc.get_sparse_core_info()
    nc, ns = info.num_cores, info.num_subcores
    nw = nc * ns
    per_w = _N_TOKENS // nw
    assert per_w % _CHUNK == 0 and per_w % _LANES == 0
    n_chunks = per_w // _CHUNK
    n_vec = per_w // _LANES
    mesh = plsc.VectorSubcoreMesh(
        core_axis_name="c", subcore_axis_name="s")

    @functools.partial(
        pl.kernel,
        out_type=jax.ShapeDtypeStruct((_N_TOKENS, _D), jnp.float32),
        mesh=mesh,
        scratch_types=[
            pltpu.VMEM((per_w,), jnp.int32),
            pltpu.VMEM((_CHUNK, _D), jnp.float32),
            pltpu.VMEM((_CHUNK, _D), jnp.float32),
            pltpu.SemaphoreType.DMA,
            pltpu.SemaphoreType.DMA,
            pltpu.SemaphoreType.DMA,
            pltpu.SemaphoreType.DMA,
        ],
    )
    def gather_kernel(table_hbm, rows_hbm, out_hbm, idx_v,
                      rows_a, rows_b, g_sem_a, g_sem_b, o_sem_a, o_sem_b):
        wid = lax.axis_index("s") * nc + lax.axis_index("c")
        base = wid * per_w
        pltpu.sync_copy(rows_hbm.at[pl.ds(base, per_w)], idx_v)
        bufs = (rows_a, rows_b)
        g_sems = (g_sem_a, g_sem_b)
        o_sems = (o_sem_a, o_sem_b)

        def start_gather(c):
            return pltpu.async_copy(
                table_hbm.at[idx_v.at[pl.ds(c * _CHUNK, _CHUNK)]],
                bufs[c % 2], g_sems[c % 2])

        def start_out(c):
            return pltpu.async_copy(
                bufs[c % 2], out_hbm.at[pl.ds(base + c * _CHUNK, _CHUNK)],
                o_sems[c % 2])

        gather_d = [None] * n_chunks
        out_d = [None] * n_chunks
        for c in range(n_chunks):
            if c >= 2:
                out_d[c - 2].wait()
            gather_d[c] = start_gather(c)
            if c >= 1:
                gather_d[c - 1].wait()
                out_d[c - 1] = start_out(c - 1)
        gather_d[n_chunks - 1].wait()
        out_d[n_chunks - 1] = start_out(n_chunks - 1)
        if n_chunks >= 2:
            out_d[n_chunks - 2].wait()
        out_d[n_chunks - 1].wait()

    return gather_kernel


def kernel(tokens, oov_features, fixed_weights):
    tokens_i32 = tokens.astype(jnp.int32)
    table = jnp.concatenate(
        [fixed_weights, oov_features.reshape(_BS * _N_OOV, _D)], axis=0)
    boff = jnp.arange(_BS, dtype=jnp.int32)[:, None] * _N_OOV
    rows = jnp.where(tokens_i32 < _VOCAB, tokens_i32,
                     tokens_i32 + boff).reshape(_N_TOKENS)
    features = _build_gather()(table, rows).reshape(_BS, _SEQ, _D)
    padding_mask = (tokens == _PAD)[:, None, None, :]
    sequential_mask = jnp.triu(jnp.ones((_SEQ, _SEQ), dtype=bool), k=1)
    return (features, (padding_mask, sequential_mask))
